# baseline (speedup 1.0000x reference)
import functools

import jax
import jax.numpy as jnp
from jax import lax
from jax.experimental import pallas as pl
from jax.experimental.pallas import tpu as pltpu
from jax.experimental.pallas import tpu_sc as plsc

HIDDEN = 64
N_CAT = 8
VOCAB = 100000
N_SCAL = 13
BATCH = 16384

TBLK = 1024
NTBLK = -(-VOCAB // TBLK)
VOCAB_PAD = NTBLK * TBLK

NC, NS, LANES = 2, 16, 16
NW = NC * NS
BPW = BATCH // NW
RPW = BPW * N_CAT
CHUNK = 128
NCHUNK = RPW // CHUNK
NVEC = RPW // LANES

X_COLS = N_CAT + N_SCAL


def _sc_gather(x, tab_flat):
    mesh = plsc.VectorSubcoreMesh(core_axis_name="c", subcore_axis_name="s")

    @functools.partial(
        pl.kernel,
        out_type=jax.ShapeDtypeStruct((BATCH * N_CAT, HIDDEN), jnp.float32),
        mesh=mesh,
        scratch_types=[
            pltpu.VMEM((BPW * X_COLS,), jnp.float32),
            pltpu.VMEM((NCHUNK, CHUNK), jnp.int32),
            pltpu.VMEM((2, CHUNK, HIDDEN), jnp.float32),
            pltpu.SemaphoreType.DMA,
            pltpu.SemaphoreType.DMA,
            pltpu.SemaphoreType.DMA,
            pltpu.SemaphoreType.DMA,
        ],
        compiler_params=pltpu.CompilerParams(
            needs_layout_passes=False, use_tc_tiling_on_sc=False
        ),
    )
    def k(x_hbm, tab_hbm, out_hbm, x_v, idx_v, rows_v, sg0, sg1, so0, so1):
        wid = lax.axis_index("s") * NC + lax.axis_index("c")
        base = wid * BPW
        obase = base * N_CAT

        pltpu.sync_copy(x_hbm.at[pl.ds(base * X_COLS, BPW * X_COLS)], x_v)

        lane = lax.iota(jnp.int32, LANES)

        @pl.loop(0, NVEC)
        def _(v):
            p = v * LANES + lane
            b = p >> 3
            t = p & 7
            val = plsc.load_gather(x_v, [b * X_COLS + t])
            idx = val.astype(jnp.int32) + t * VOCAB_PAD
            idx_v[v >> 3, pl.ds((v & 7) * LANES, LANES)] = idx

        sg = (sg0, sg1)
        so = (so0, so1)

        def g_start(g, s):
            pltpu.async_copy(tab_hbm.at[idx_v.at[g]], rows_v.at[s], sg[s])

        def g_wait(s):
            pltpu.make_async_copy(
                tab_hbm.at[idx_v.at[0]], rows_v.at[s], sg[s]).wait()

        def o_start(g, s):
            pltpu.async_copy(
                rows_v.at[s], out_hbm.at[pl.ds(obase + g * CHUNK, CHUNK)], so[s])

        def o_wait(s):
            pltpu.make_async_copy(
                rows_v.at[s], out_hbm.at[pl.ds(obase, CHUNK)], so[s]).wait()

        g_start(0, 0)
        g_start(1, 1)

        @pl.loop(0, NCHUNK - 2, step=2)
        def _(g):
            for s in (0, 1):
                gg = g + s
                g_wait(s)
                o_start(gg, s)
                o_wait(s)
                g_start(gg + 2, s)

        for s, gg in ((0, NCHUNK - 2), (1, NCHUNK - 1)):
            g_wait(s)
            o_start(gg, s)
        for s in (0, 1):
            o_wait(s)

    return k(x, tab_flat)


def _tc_transpose(tt):
    def body(in_ref, out_ref):
        a = in_ref[0]
        tr = jnp.transpose(a)
        tr3 = tr.reshape(TBLK // 2, 2, HIDDEN)
        out_ref[0] = jnp.concatenate([tr3[:, 0, :], tr3[:, 1, :]], axis=-1)

    return pl.pallas_call(
        body,
        grid=(N_CAT, NTBLK),
        in_specs=[pl.BlockSpec((1, HIDDEN, TBLK), lambda t, w: (t, 0, w))],
        out_specs=pl.BlockSpec((1, TBLK // 2, 2 * HIDDEN), lambda t, w: (t, w, 0)),
        out_shape=jax.ShapeDtypeStruct(
            (N_CAT, VOCAB_PAD // 2, 2 * HIDDEN), jnp.float32
        ),
    )(tt)


def _tc_project(g2, xs, wd1, wl, bl2, wd2, bd2):
    BM = 2048

    def body(g_ref, xs_ref, wd1_ref, wl_ref, bl_ref, wd2_ref, bd_ref, o_ref):
        scal = (
            jnp.dot(xs_ref[...], wl_ref[...], preferred_element_type=jnp.float32)
            + bl_ref[...]
        )
        acc = jnp.dot(g_ref[...], wd1_ref[...], preferred_element_type=jnp.float32)
        acc = acc + jnp.dot(scal, wd2_ref[...], preferred_element_type=jnp.float32)
        o_ref[...] = acc + bd_ref[...]

    d1 = N_CAT * HIDDEN
    return pl.pallas_call(
        body,
        grid=(BATCH // BM,),
        in_specs=[
            pl.BlockSpec((BM, d1), lambda i: (i, 0)),
            pl.BlockSpec((BM, N_SCAL), lambda i: (i, 0)),
            pl.BlockSpec((d1, HIDDEN), lambda i: (0, 0)),
            pl.BlockSpec((N_SCAL, HIDDEN), lambda i: (0, 0)),
            pl.BlockSpec((1, HIDDEN), lambda i: (0, 0)),
            pl.BlockSpec((HIDDEN, HIDDEN), lambda i: (0, 0)),
            pl.BlockSpec((1, HIDDEN), lambda i: (0, 0)),
        ],
        out_specs=pl.BlockSpec((BM, HIDDEN), lambda i: (i, 0)),
        out_shape=jax.ShapeDtypeStruct((BATCH, HIDDEN), jnp.float32),
    )(g2, xs, wd1, wl, bl2, wd2, bd2)


def kernel(x, tables, Wl, bl, Wd, bd):
    tt = tables.transpose(0, 2, 1)
    t2 = _tc_transpose(tt)
    tab_flat = t2.reshape(N_CAT * VOCAB_PAD, HIDDEN)
    gathered = _sc_gather(x.reshape(-1), tab_flat)
    g2 = gathered.reshape(BATCH, N_CAT * HIDDEN)
    xs = x[:, N_CAT:]
    wd1 = Wd[: N_CAT * HIDDEN]
    wd2 = Wd[N_CAT * HIDDEN :]
    return _tc_project(
        g2, xs, wd1, Wl, bl.reshape(1, HIDDEN), wd2, bd.reshape(1, HIDDEN)
    )


# speedup vs baseline: 1.4459x; 1.4459x over previous
import functools

import jax
import jax.numpy as jnp
from jax import lax
from jax.experimental import pallas as pl
from jax.experimental.pallas import tpu as pltpu
from jax.experimental.pallas import tpu_sc as plsc

HIDDEN = 64
N_CAT = 8
VOCAB = 100000
N_SCAL = 13
BATCH = 16384

TBLK = 1024
NTBLK = -(-VOCAB // TBLK)
VOCAB_PAD = NTBLK * TBLK
NPAIR = N_CAT // 2

NC, NS, LANES = 2, 16, 16
NW = NC * NS
BPW = BATCH // NW
RPW = BPW * N_CAT
CHUNK = 128
NCHUNK = RPW // CHUNK
SUBC = BPW // CHUNK
NVEC = RPW // LANES

X_COLS = N_CAT + N_SCAL


def _tc_transpose(tt):
    def body(a_ref, b_ref, out_ref):
        out_ref[0] = jnp.concatenate(
            [jnp.transpose(a_ref[0]), jnp.transpose(b_ref[0])], axis=-1
        )

    return pl.pallas_call(
        body,
        grid=(NPAIR, NTBLK),
        in_specs=[
            pl.BlockSpec((1, HIDDEN, TBLK), lambda g, w: (2 * g, 0, w)),
            pl.BlockSpec((1, HIDDEN, TBLK), lambda g, w: (2 * g + 1, 0, w)),
        ],
        out_specs=pl.BlockSpec((1, TBLK, 2 * HIDDEN), lambda g, w: (g, w, 0)),
        out_shape=jax.ShapeDtypeStruct((NPAIR, VOCAB_PAD, 2 * HIDDEN), jnp.float32),
    )(tt, tt)


def _sc_gather(x, tab_flat):
    mesh = plsc.VectorSubcoreMesh(core_axis_name="c", subcore_axis_name="s")

    @functools.partial(
        pl.kernel,
        out_type=jax.ShapeDtypeStruct((BATCH * N_CAT, 2 * HIDDEN), jnp.float32),
        mesh=mesh,
        scratch_types=[
            pltpu.VMEM((BPW * X_COLS,), jnp.float32),
            pltpu.VMEM((NCHUNK, CHUNK), jnp.int32),
            pltpu.VMEM((2, CHUNK, 2 * HIDDEN), jnp.float32),
            pltpu.SemaphoreType.DMA,
            pltpu.SemaphoreType.DMA,
            pltpu.SemaphoreType.DMA,
            pltpu.SemaphoreType.DMA,
        ],
        compiler_params=pltpu.CompilerParams(
            needs_layout_passes=False, use_tc_tiling_on_sc=False
        ),
    )
    def k(x_hbm, tab_hbm, out_hbm, x_v, idx_v, rows_v, sg0, sg1, so0, so1):
        wid = lax.axis_index("s") * NC + lax.axis_index("c")
        base = wid * BPW

        pltpu.sync_copy(x_hbm.at[pl.ds(base * X_COLS, BPW * X_COLS)], x_v)

        lane = lax.iota(jnp.int32, LANES)

        @pl.loop(0, NVEC)
        def _(v):
            p = v * LANES + lane
            q = p >> 9
            b = p & (BPW - 1)
            val = plsc.load_gather(x_v, [b * X_COLS + q])
            idx = val.astype(jnp.int32) + (q >> 1) * VOCAB_PAD
            idx_v[v >> 3, pl.ds((v & 7) * LANES, LANES)] = idx

        sg = (sg0, sg1)
        so = (so0, so1)

        def out_off(g):
            return (g >> 2) * BATCH + base + (g & (SUBC - 1)) * CHUNK

        def g_start(g, s):
            pltpu.async_copy(tab_hbm.at[idx_v.at[g]], rows_v.at[s], sg[s])

        def g_wait(s):
            pltpu.make_async_copy(
                tab_hbm.at[idx_v.at[0]], rows_v.at[s], sg[s]).wait()

        def o_start(g, s):
            pltpu.async_copy(
                rows_v.at[s], out_hbm.at[pl.ds(out_off(g), CHUNK)], so[s])

        def o_wait(s):
            pltpu.make_async_copy(
                rows_v.at[s], out_hbm.at[pl.ds(0, CHUNK)], so[s]).wait()

        g_start(0, 0)
        g_start(1, 1)

        @pl.loop(0, NCHUNK - 2, step=2)
        def _(g):
            for s in (0, 1):
                gg = g + s
                g_wait(s)
                o_start(gg, s)
                o_wait(s)
                g_start(gg + 2, s)

        for s, gg in ((0, NCHUNK - 2), (1, NCHUNK - 1)):
            g_wait(s)
            o_start(gg, s)
        for s in (0, 1):
            o_wait(s)

    return k(x, tab_flat)


def _tc_project(g3, w3, xs, wl, bl2, wd2, bd2):
    BM = 2048

    def body(g_ref, w_ref, xs_ref, wl_ref, bl_ref, wd2_ref, bd_ref, o_ref):
        q = pl.program_id(1)

        @pl.when(q == 0)
        def _():
            scal = (
                jnp.dot(xs_ref[...], wl_ref[...],
                        preferred_element_type=jnp.float32)
                + bl_ref[...]
            )
            o_ref[...] = (
                jnp.dot(scal, wd2_ref[...], preferred_element_type=jnp.float32)
                + bd_ref[...]
            )

        o_ref[...] += jnp.dot(
            g_ref[0], w_ref[0], preferred_element_type=jnp.float32
        )

    return pl.pallas_call(
        body,
        grid=(BATCH // BM, N_CAT),
        in_specs=[
            pl.BlockSpec((1, BM, 2 * HIDDEN), lambda i, q: (q, i, 0)),
            pl.BlockSpec((1, 2 * HIDDEN, HIDDEN), lambda i, q: (q, 0, 0)),
            pl.BlockSpec((BM, N_SCAL), lambda i, q: (i, 0)),
            pl.BlockSpec((N_SCAL, HIDDEN), lambda i, q: (0, 0)),
            pl.BlockSpec((1, HIDDEN), lambda i, q: (0, 0)),
            pl.BlockSpec((HIDDEN, HIDDEN), lambda i, q: (0, 0)),
            pl.BlockSpec((1, HIDDEN), lambda i, q: (0, 0)),
        ],
        out_specs=pl.BlockSpec((BM, HIDDEN), lambda i, q: (i, 0)),
        out_shape=jax.ShapeDtypeStruct((BATCH, HIDDEN), jnp.float32),
    )(g3, w3, xs, wl, bl2, wd2, bd2)


def kernel(x, tables, Wl, bl, Wd, bd):
    tt = tables.transpose(0, 2, 1)
    t2 = _tc_transpose(tt)
    tab_flat = t2.reshape(NPAIR * VOCAB_PAD, 2 * HIDDEN)
    gathered = _sc_gather(x.reshape(-1), tab_flat)
    g3 = gathered.reshape(N_CAT, BATCH, 2 * HIDDEN)

    wd1 = Wd[: N_CAT * HIDDEN].reshape(N_CAT, HIDDEN, HIDDEN)
    w3 = jnp.zeros((N_CAT, 2, HIDDEN, HIDDEN), jnp.float32)
    w3 = w3.at[jnp.arange(N_CAT), jnp.arange(N_CAT) % 2].set(wd1)
    w3 = w3.reshape(N_CAT, 2 * HIDDEN, HIDDEN)

    xs = x[:, N_CAT:]
    wd2 = Wd[N_CAT * HIDDEN :]
    return _tc_project(
        g3, w3, xs, Wl, bl.reshape(1, HIDDEN), wd2, bd.reshape(1, HIDDEN)
    )


# speedup vs baseline: 1.5576x; 1.0773x over previous
import functools

import jax
import jax.numpy as jnp
from jax import lax
from jax.experimental import pallas as pl
from jax.experimental.pallas import tpu as pltpu
from jax.experimental.pallas import tpu_sc as plsc

HIDDEN = 64
N_CAT = 8
VOCAB = 100000
N_SCAL = 13
BATCH = 16384

TBLK = 1024
NTBLK = -(-VOCAB // TBLK)
VOCAB_PAD = NTBLK * TBLK
NPAIR = N_CAT // 2

NC, NS, LANES = 2, 16, 16
NW = NC * NS
BPW = BATCH // NW
RPW = BPW * N_CAT
CHUNK = 128
NCHUNK = RPW // CHUNK
SUBC = BPW // CHUNK
NVEC = RPW // LANES

X_COLS = N_CAT + N_SCAL


def _tc_transpose(tt4):
    def body(a_ref, out_ref):
        i = lax.broadcasted_iota(jnp.int32, (2 * HIDDEN, 2 * HIDDEN), 0)
        j = lax.broadcasted_iota(jnp.int32, (2 * HIDDEN, 2 * HIDDEN), 1)
        eye = (i == j).astype(jnp.float32)
        out_ref[0] = lax.dot_general(
            a_ref[0], eye, (((0,), (0,)), ((), ())),
            preferred_element_type=jnp.float32,
        )

    return pl.pallas_call(
        body,
        grid=(NPAIR, NTBLK),
        in_specs=[pl.BlockSpec((1, 2 * HIDDEN, TBLK), lambda g, w: (g, 0, w))],
        out_specs=pl.BlockSpec((1, TBLK, 2 * HIDDEN), lambda g, w: (g, w, 0)),
        out_shape=jax.ShapeDtypeStruct((NPAIR, VOCAB_PAD, 2 * HIDDEN), jnp.float32),
    )(tt4)


def _sc_gather(x, tab_flat):
    mesh = plsc.VectorSubcoreMesh(core_axis_name="c", subcore_axis_name="s")

    @functools.partial(
        pl.kernel,
        out_type=jax.ShapeDtypeStruct((BATCH * N_CAT, 2 * HIDDEN), jnp.float32),
        mesh=mesh,
        scratch_types=[
            pltpu.VMEM((BPW * X_COLS,), jnp.float32),
            pltpu.VMEM((NCHUNK, CHUNK), jnp.int32),
            pltpu.VMEM((2, CHUNK, 2 * HIDDEN), jnp.float32),
            pltpu.SemaphoreType.DMA,
            pltpu.SemaphoreType.DMA,
            pltpu.SemaphoreType.DMA,
            pltpu.SemaphoreType.DMA,
        ],
        compiler_params=pltpu.CompilerParams(
            needs_layout_passes=False, use_tc_tiling_on_sc=False
        ),
    )
    def k(x_hbm, tab_hbm, out_hbm, x_v, idx_v, rows_v, sg0, sg1, so0, so1):
        wid = lax.axis_index("s") * NC + lax.axis_index("c")
        base = wid * BPW

        pltpu.sync_copy(x_hbm.at[pl.ds(base * X_COLS, BPW * X_COLS)], x_v)

        lane = lax.iota(jnp.int32, LANES)

        @pl.loop(0, NVEC)
        def _(v):
            p = v * LANES + lane
            q = p >> 9
            b = p & (BPW - 1)
            val = plsc.load_gather(x_v, [b * X_COLS + q])
            idx = val.astype(jnp.int32) + (q >> 1) * VOCAB_PAD
            idx_v[v >> 3, pl.ds((v & 7) * LANES, LANES)] = idx

        sg = (sg0, sg1)
        so = (so0, so1)

        def out_off(g):
            return (g >> 2) * BATCH + base + (g & (SUBC - 1)) * CHUNK

        def g_start(g, s):
            pltpu.async_copy(tab_hbm.at[idx_v.at[g]], rows_v.at[s], sg[s])

        def g_wait(s):
            pltpu.make_async_copy(
                tab_hbm.at[idx_v.at[0]], rows_v.at[s], sg[s]).wait()

        def o_start(g, s):
            pltpu.async_copy(
                rows_v.at[s], out_hbm.at[pl.ds(out_off(g), CHUNK)], so[s])

        def o_wait(s):
            pltpu.make_async_copy(
                rows_v.at[s], out_hbm.at[pl.ds(0, CHUNK)], so[s]).wait()

        g_start(0, 0)
        g_start(1, 1)

        @pl.loop(0, NCHUNK - 2, step=2)
        def _(g):
            for s in (0, 1):
                gg = g + s
                g_wait(s)
                o_start(gg, s)
                o_wait(s)
                g_start(gg + 2, s)

        for s, gg in ((0, NCHUNK - 2), (1, NCHUNK - 1)):
            g_wait(s)
            o_start(gg, s)
        for s in (0, 1):
            o_wait(s)

    return k(x, tab_flat)


def _tc_project(g3, w3, xs, wl, bl2, wd2, bd2):
    BM = 2048

    def body(g_ref, w_ref, xs_ref, wl_ref, bl_ref, wd2_ref, bd_ref, o_ref):
        q = pl.program_id(1)

        @pl.when(q == 0)
        def _():
            scal = (
                jnp.dot(xs_ref[...], wl_ref[...],
                        preferred_element_type=jnp.float32)
                + bl_ref[...]
            )
            o_ref[...] = (
                jnp.dot(scal, wd2_ref[...], preferred_element_type=jnp.float32)
                + bd_ref[...]
            )

        o_ref[...] += jnp.dot(
            g_ref[0], w_ref[0], preferred_element_type=jnp.float32
        )

    return pl.pallas_call(
        body,
        grid=(BATCH // BM, N_CAT),
        in_specs=[
            pl.BlockSpec((1, BM, 2 * HIDDEN), lambda i, q: (q, i, 0)),
            pl.BlockSpec((1, 2 * HIDDEN, HIDDEN), lambda i, q: (q, 0, 0)),
            pl.BlockSpec((BM, N_SCAL), lambda i, q: (i, 0)),
            pl.BlockSpec((N_SCAL, HIDDEN), lambda i, q: (0, 0)),
            pl.BlockSpec((1, HIDDEN), lambda i, q: (0, 0)),
            pl.BlockSpec((HIDDEN, HIDDEN), lambda i, q: (0, 0)),
            pl.BlockSpec((1, HIDDEN), lambda i, q: (0, 0)),
        ],
        out_specs=pl.BlockSpec((BM, HIDDEN), lambda i, q: (i, 0)),
        out_shape=jax.ShapeDtypeStruct((BATCH, HIDDEN), jnp.float32),
    )(g3, w3, xs, wl, bl2, wd2, bd2)


def kernel(x, tables, Wl, bl, Wd, bd):
    tt = tables.transpose(0, 2, 1)
    tt4 = tt.reshape(NPAIR, 2 * HIDDEN, VOCAB)
    t2 = _tc_transpose(tt4)
    tab_flat = t2.reshape(NPAIR * VOCAB_PAD, 2 * HIDDEN)
    gathered = _sc_gather(x.reshape(-1), tab_flat)
    g3 = gathered.reshape(N_CAT, BATCH, 2 * HIDDEN)

    wd1 = Wd[: N_CAT * HIDDEN].reshape(N_CAT, HIDDEN, HIDDEN)
    w3 = jnp.zeros((N_CAT, 2, HIDDEN, HIDDEN), jnp.float32)
    w3 = w3.at[jnp.arange(N_CAT), jnp.arange(N_CAT) % 2].set(wd1)
    w3 = w3.reshape(N_CAT, 2 * HIDDEN, HIDDEN)

    xs = x[:, N_CAT:]
    wd2 = Wd[N_CAT * HIDDEN :]
    return _tc_project(
        g3, w3, xs, Wl, bl.reshape(1, HIDDEN), wd2, bd.reshape(1, HIDDEN)
    )


# speedup vs baseline: 1.9540x; 1.2545x over previous
import functools

import jax
import jax.numpy as jnp
from jax import lax
from jax.experimental import pallas as pl
from jax.experimental.pallas import tpu as pltpu
from jax.experimental.pallas import tpu_sc as plsc

HIDDEN = 64
N_CAT = 8
VOCAB = 100000
N_SCAL = 13
BATCH = 16384

TBLK = 2048
NTBLK = -(-VOCAB // TBLK)
VOCAB_PAD = NTBLK * TBLK
NPAIR = N_CAT // 2

NC, NS, LANES = 2, 16, 16
NW = NC * NS
BPW = BATCH // NW
RPW = BPW * 2
CHUNK = 128
NCHUNK = RPW // CHUNK
SUBC = BPW // CHUNK
NVEC = RPW // LANES

X_COLS = N_CAT + N_SCAL


def _tc_transpose(tt4, g):
    def body(a_ref, out_ref):
        i = lax.broadcasted_iota(jnp.int32, (2 * HIDDEN, 2 * HIDDEN), 0)
        j = lax.broadcasted_iota(jnp.int32, (2 * HIDDEN, 2 * HIDDEN), 1)
        eye = (i == j).astype(jnp.float32)
        out_ref[...] = lax.dot_general(
            a_ref[0], eye, (((0,), (0,)), ((), ())),
            preferred_element_type=jnp.float32,
        )

    return pl.pallas_call(
        body,
        grid=(NTBLK,),
        in_specs=[pl.BlockSpec((1, 2 * HIDDEN, TBLK), lambda w: (g, 0, w))],
        out_specs=pl.BlockSpec((TBLK, 2 * HIDDEN), lambda w: (w, 0)),
        out_shape=jax.ShapeDtypeStruct((VOCAB_PAD, 2 * HIDDEN), jnp.float32),
    )(tt4)


def _sc_gather(x, t2, g):
    mesh = plsc.VectorSubcoreMesh(core_axis_name="c", subcore_axis_name="s")

    @functools.partial(
        pl.kernel,
        out_type=jax.ShapeDtypeStruct((2 * BATCH, 2 * HIDDEN), jnp.float32),
        mesh=mesh,
        scratch_types=[
            pltpu.VMEM((BPW * X_COLS,), jnp.float32),
            pltpu.VMEM((NCHUNK, CHUNK), jnp.int32),
            pltpu.VMEM((2, CHUNK, 2 * HIDDEN), jnp.float32),
            pltpu.SemaphoreType.DMA,
            pltpu.SemaphoreType.DMA,
            pltpu.SemaphoreType.DMA,
            pltpu.SemaphoreType.DMA,
        ],
        compiler_params=pltpu.CompilerParams(
            needs_layout_passes=False, use_tc_tiling_on_sc=False
        ),
    )
    def k(x_hbm, tab_hbm, out_hbm, x_v, idx_v, rows_v, sg0, sg1, so0, so1):
        wid = lax.axis_index("s") * NC + lax.axis_index("c")
        base = wid * BPW

        pltpu.sync_copy(x_hbm.at[pl.ds(base * X_COLS, BPW * X_COLS)], x_v)

        lane = lax.iota(jnp.int32, LANES)

        @pl.loop(0, NVEC)
        def _(v):
            p = v * LANES + lane
            q = p >> 9
            b = p & (BPW - 1)
            val = plsc.load_gather(x_v, [b * X_COLS + (2 * g + q)])
            idx_v[v >> 3, pl.ds((v & 7) * LANES, LANES)] = val.astype(jnp.int32)

        sg = (sg0, sg1)
        so = (so0, so1)

        def out_off(gg):
            return (gg >> 2) * BATCH + base + (gg & (SUBC - 1)) * CHUNK

        def g_start(gg, s):
            pltpu.async_copy(tab_hbm.at[idx_v.at[gg]], rows_v.at[s], sg[s])

        def g_wait(s):
            pltpu.make_async_copy(
                tab_hbm.at[idx_v.at[0]], rows_v.at[s], sg[s]).wait()

        def o_start(gg, s):
            pltpu.async_copy(
                rows_v.at[s], out_hbm.at[pl.ds(out_off(gg), CHUNK)], so[s])

        def o_wait(s):
            pltpu.make_async_copy(
                rows_v.at[s], out_hbm.at[pl.ds(0, CHUNK)], so[s]).wait()

        g_start(0, 0)
        g_start(1, 1)

        @pl.loop(0, NCHUNK - 2, step=2)
        def _(gg):
            for s in (0, 1):
                g_wait(s)
                o_start(gg + s, s)
                o_wait(s)
                g_start(gg + s + 2, s)

        for s, gg in ((0, NCHUNK - 2), (1, NCHUNK - 1)):
            g_wait(s)
            o_start(gg, s)
        for s in (0, 1):
            o_wait(s)

    return k(x, t2)


def _tc_project_first(g3, w3, xs, wl, bl2, wd2, bd2):
    BM = 2048

    def body(g_ref, w_ref, xs_ref, wl_ref, bl_ref, wd2_ref, bd_ref, o_ref):
        q = pl.program_id(1)

        @pl.when(q == 0)
        def _():
            scal = (
                jnp.dot(xs_ref[...], wl_ref[...],
                        preferred_element_type=jnp.float32)
                + bl_ref[...]
            )
            o_ref[...] = (
                jnp.dot(scal, wd2_ref[...], preferred_element_type=jnp.float32)
                + bd_ref[...]
            )

        o_ref[...] += jnp.dot(
            g_ref[0], w_ref[0], preferred_element_type=jnp.float32
        )

    return pl.pallas_call(
        body,
        grid=(BATCH // BM, 2),
        in_specs=[
            pl.BlockSpec((1, BM, 2 * HIDDEN), lambda i, q: (q, i, 0)),
            pl.BlockSpec((1, 2 * HIDDEN, HIDDEN), lambda i, q: (q, 0, 0)),
            pl.BlockSpec((BM, N_SCAL), lambda i, q: (i, 0)),
            pl.BlockSpec((N_SCAL, HIDDEN), lambda i, q: (0, 0)),
            pl.BlockSpec((1, HIDDEN), lambda i, q: (0, 0)),
            pl.BlockSpec((HIDDEN, HIDDEN), lambda i, q: (0, 0)),
            pl.BlockSpec((1, HIDDEN), lambda i, q: (0, 0)),
        ],
        out_specs=pl.BlockSpec((BM, HIDDEN), lambda i, q: (i, 0)),
        out_shape=jax.ShapeDtypeStruct((BATCH, HIDDEN), jnp.float32),
    )(g3, w3, xs, wl, bl2, wd2, bd2)


def _tc_project_next(g3, w3, prev):
    BM = 2048

    def body(g_ref, w_ref, p_ref, o_ref):
        q = pl.program_id(1)

        @pl.when(q == 0)
        def _():
            o_ref[...] = p_ref[...]

        o_ref[...] += jnp.dot(
            g_ref[0], w_ref[0], preferred_element_type=jnp.float32
        )

    return pl.pallas_call(
        body,
        grid=(BATCH // BM, 2),
        in_specs=[
            pl.BlockSpec((1, BM, 2 * HIDDEN), lambda i, q: (q, i, 0)),
            pl.BlockSpec((1, 2 * HIDDEN, HIDDEN), lambda i, q: (q, 0, 0)),
            pl.BlockSpec((BM, HIDDEN), lambda i, q: (i, 0)),
        ],
        out_specs=pl.BlockSpec((BM, HIDDEN), lambda i, q: (i, 0)),
        out_shape=jax.ShapeDtypeStruct((BATCH, HIDDEN), jnp.float32),
    )(g3, w3, prev)


def kernel(x, tables, Wl, bl, Wd, bd):
    tt = tables.transpose(0, 2, 1)
    tt4 = tt.reshape(NPAIR, 2 * HIDDEN, VOCAB)
    x_flat = x.reshape(-1)

    wd1 = Wd[: N_CAT * HIDDEN].reshape(N_CAT, HIDDEN, HIDDEN)
    w3 = jnp.zeros((N_CAT, 2, HIDDEN, HIDDEN), jnp.float32)
    w3 = w3.at[jnp.arange(N_CAT), jnp.arange(N_CAT) % 2].set(wd1)
    w3 = w3.reshape(N_CAT, 2 * HIDDEN, HIDDEN)

    xs = x[:, N_CAT:]
    wd2 = Wd[N_CAT * HIDDEN :]

    acc = None
    for g in range(NPAIR):
        t2 = _tc_transpose(tt4, g)
        gathered = _sc_gather(x_flat, t2, g)
        g3 = gathered.reshape(2, BATCH, 2 * HIDDEN)
        w3g = w3[2 * g : 2 * g + 2]
        if acc is None:
            acc = _tc_project_first(
                g3, w3g, xs, Wl, bl.reshape(1, HIDDEN), wd2,
                bd.reshape(1, HIDDEN),
            )
        else:
            acc = _tc_project_next(g3, w3g, acc)
    return acc


# speedup vs baseline: 2.2703x; 1.1619x over previous
import functools

import jax
import jax.numpy as jnp
from jax import lax
from jax.experimental import pallas as pl
from jax.experimental.pallas import tpu as pltpu
from jax.experimental.pallas import tpu_sc as plsc

HIDDEN = 64
N_CAT = 8
VOCAB = 100000
N_SCAL = 13
BATCH = 16384

TBLK = 4096
NTBLK = -(-VOCAB // TBLK)
VOCAB_PAD = NTBLK * TBLK
NPAIR = N_CAT // 2

NC, NS, LANES = 2, 16, 16
NW = NC * NS
BPW = BATCH // NW
RPW = BPW * 2
CHUNK = 128
NCHUNK = RPW // CHUNK
SUBC = BPW // CHUNK
NVEC = RPW // LANES

X_COLS = N_CAT + N_SCAL


def _tc_transpose(tt4, g):
    def body(a_ref, out_ref):
        i = lax.broadcasted_iota(jnp.int32, (2 * HIDDEN, 2 * HIDDEN), 0)
        j = lax.broadcasted_iota(jnp.int32, (2 * HIDDEN, 2 * HIDDEN), 1)
        eye = (i == j).astype(jnp.float32)
        out_ref[...] = lax.dot_general(
            a_ref[0], eye, (((0,), (0,)), ((), ())),
            preferred_element_type=jnp.float32,
        )

    return pl.pallas_call(
        body,
        grid=(NTBLK,),
        in_specs=[pl.BlockSpec((1, 2 * HIDDEN, TBLK), lambda w: (g, 0, w))],
        out_specs=pl.BlockSpec((TBLK, 2 * HIDDEN), lambda w: (w, 0)),
        out_shape=jax.ShapeDtypeStruct((VOCAB_PAD, 2 * HIDDEN), jnp.float32),
    )(tt4)


def _sc_gather(x, t2, g):
    mesh = plsc.VectorSubcoreMesh(core_axis_name="c", subcore_axis_name="s")

    @functools.partial(
        pl.kernel,
        out_type=jax.ShapeDtypeStruct((2 * BATCH, 2 * HIDDEN), jnp.float32),
        mesh=mesh,
        scratch_types=[
            pltpu.VMEM((BPW * X_COLS,), jnp.float32),
            pltpu.VMEM((NCHUNK, CHUNK), jnp.int32),
            pltpu.VMEM((2, CHUNK, 2 * HIDDEN), jnp.float32),
            pltpu.SemaphoreType.DMA,
            pltpu.SemaphoreType.DMA,
            pltpu.SemaphoreType.DMA,
            pltpu.SemaphoreType.DMA,
        ],
        compiler_params=pltpu.CompilerParams(
            needs_layout_passes=False, use_tc_tiling_on_sc=False
        ),
    )
    def k(x_hbm, tab_hbm, out_hbm, x_v, idx_v, rows_v, sg0, sg1, so0, so1):
        wid = lax.axis_index("s") * NC + lax.axis_index("c")
        base = wid * BPW

        pltpu.sync_copy(x_hbm.at[pl.ds(base * X_COLS, BPW * X_COLS)], x_v)

        lane = lax.iota(jnp.int32, LANES)

        @pl.loop(0, NVEC)
        def _(v):
            p = v * LANES + lane
            q = p >> 9
            b = p & (BPW - 1)
            val = plsc.load_gather(x_v, [b * X_COLS + (2 * g + q)])
            idx_v[v >> 3, pl.ds((v & 7) * LANES, LANES)] = val.astype(jnp.int32)

        sg = (sg0, sg1)
        so = (so0, so1)

        def out_off(gg):
            return (gg >> 2) * BATCH + base + (gg & (SUBC - 1)) * CHUNK

        def g_start(gg, s):
            pltpu.async_copy(tab_hbm.at[idx_v.at[gg]], rows_v.at[s], sg[s])

        def g_wait(s):
            pltpu.make_async_copy(
                tab_hbm.at[idx_v.at[0]], rows_v.at[s], sg[s]).wait()

        def o_start(gg, s):
            pltpu.async_copy(
                rows_v.at[s], out_hbm.at[pl.ds(out_off(gg), CHUNK)], so[s])

        def o_wait(s):
            pltpu.make_async_copy(
                rows_v.at[s], out_hbm.at[pl.ds(0, CHUNK)], so[s]).wait()

        g_start(0, 0)
        g_start(1, 1)

        @pl.loop(0, NCHUNK - 2, step=2)
        def _(gg):
            for s in (0, 1):
                g_wait(s)
                o_start(gg + s, s)
                o_wait(s)
                g_start(gg + s + 2, s)

        for s, gg in ((0, NCHUNK - 2), (1, NCHUNK - 1)):
            g_wait(s)
            o_start(gg, s)
        for s in (0, 1):
            o_wait(s)

    return k(x, t2)


def _tc_project_first(g3, w3, xs, wl, bl2, wd2, bd2):
    BM = 2048

    def body(g_ref, w_ref, xs_ref, wl_ref, bl_ref, wd2_ref, bd_ref, o_ref):
        q = pl.program_id(1)

        @pl.when(q == 0)
        def _():
            scal = (
                jnp.dot(xs_ref[...], wl_ref[...],
                        preferred_element_type=jnp.float32)
                + bl_ref[...]
            )
            o_ref[...] = (
                jnp.dot(scal, wd2_ref[...], preferred_element_type=jnp.float32)
                + bd_ref[...]
            )

        o_ref[...] += jnp.dot(
            g_ref[0], w_ref[0], preferred_element_type=jnp.float32
        )

    return pl.pallas_call(
        body,
        grid=(BATCH // BM, 2),
        in_specs=[
            pl.BlockSpec((1, BM, 2 * HIDDEN), lambda i, q: (q, i, 0)),
            pl.BlockSpec((1, 2 * HIDDEN, HIDDEN), lambda i, q: (q, 0, 0)),
            pl.BlockSpec((BM, N_SCAL), lambda i, q: (i, 0)),
            pl.BlockSpec((N_SCAL, HIDDEN), lambda i, q: (0, 0)),
            pl.BlockSpec((1, HIDDEN), lambda i, q: (0, 0)),
            pl.BlockSpec((HIDDEN, HIDDEN), lambda i, q: (0, 0)),
            pl.BlockSpec((1, HIDDEN), lambda i, q: (0, 0)),
        ],
        out_specs=pl.BlockSpec((BM, HIDDEN), lambda i, q: (i, 0)),
        out_shape=jax.ShapeDtypeStruct((BATCH, HIDDEN), jnp.float32),
    )(g3, w3, xs, wl, bl2, wd2, bd2)


def _tc_project_next(g3, w3, prev):
    BM = 2048

    def body(g_ref, w_ref, p_ref, o_ref):
        q = pl.program_id(1)

        @pl.when(q == 0)
        def _():
            o_ref[...] = p_ref[...]

        o_ref[...] += jnp.dot(
            g_ref[0], w_ref[0], preferred_element_type=jnp.float32
        )

    return pl.pallas_call(
        body,
        grid=(BATCH // BM, 2),
        in_specs=[
            pl.BlockSpec((1, BM, 2 * HIDDEN), lambda i, q: (q, i, 0)),
            pl.BlockSpec((1, 2 * HIDDEN, HIDDEN), lambda i, q: (q, 0, 0)),
            pl.BlockSpec((BM, HIDDEN), lambda i, q: (i, 0)),
        ],
        out_specs=pl.BlockSpec((BM, HIDDEN), lambda i, q: (i, 0)),
        out_shape=jax.ShapeDtypeStruct((BATCH, HIDDEN), jnp.float32),
    )(g3, w3, prev)


def kernel(x, tables, Wl, bl, Wd, bd):
    tt = tables.transpose(0, 2, 1)
    tt4 = tt.reshape(NPAIR, 2 * HIDDEN, VOCAB)
    x_flat = x.reshape(-1)

    wd1 = Wd[: N_CAT * HIDDEN].reshape(N_CAT, HIDDEN, HIDDEN)
    w3 = jnp.zeros((N_CAT, 2, HIDDEN, HIDDEN), jnp.float32)
    w3 = w3.at[jnp.arange(N_CAT), jnp.arange(N_CAT) % 2].set(wd1)
    w3 = w3.reshape(N_CAT, 2 * HIDDEN, HIDDEN)
    xs = x[:, N_CAT:]
    wd2 = Wd[N_CAT * HIDDEN :]

    acc = None
    for g in range(NPAIR):
        t2 = _tc_transpose(tt4, g)
        gathered = _sc_gather(x_flat, t2, g)
        g3 = gathered.reshape(2, BATCH, 2 * HIDDEN)
        w3g = w3[2 * g : 2 * g + 2]
        if acc is None:
            acc = _tc_project_first(
                g3, w3g, xs, Wl, bl.reshape(1, HIDDEN), wd2,
                bd.reshape(1, HIDDEN),
            )
        else:
            acc = _tc_project_next(g3, w3g, acc)
    return acc


# speedup vs baseline: 2.3966x; 1.0557x over previous
import functools

import jax
import jax.numpy as jnp
from jax import lax
from jax.experimental import pallas as pl
from jax.experimental.pallas import tpu as pltpu
from jax.experimental.pallas import tpu_sc as plsc

HIDDEN = 64
N_CAT = 8
VOCAB = 100000
N_SCAL = 13
BATCH = 16384

TBLK = 8192
NTBLK = -(-VOCAB // TBLK)
VOCAB_PAD = NTBLK * TBLK
NPAIR = N_CAT // 2

NC, NS, LANES = 2, 16, 16
NW = NC * NS
BPW = BATCH // NW
RPW = BPW * 2
CHUNK = 128
NCHUNK = RPW // CHUNK
SUBC = BPW // CHUNK
NVEC = RPW // LANES

X_COLS = N_CAT + N_SCAL


def _tc_transpose(tt4, g):
    def body(a_ref, out_ref):
        i = lax.broadcasted_iota(jnp.int32, (2 * HIDDEN, 2 * HIDDEN), 0)
        j = lax.broadcasted_iota(jnp.int32, (2 * HIDDEN, 2 * HIDDEN), 1)
        eye = (i == j).astype(jnp.float32)
        out_ref[...] = lax.dot_general(
            a_ref[0], eye, (((0,), (0,)), ((), ())),
            preferred_element_type=jnp.float32,
        )

    return pl.pallas_call(
        body,
        grid=(NTBLK,),
        in_specs=[pl.BlockSpec((1, 2 * HIDDEN, TBLK), lambda w: (g, 0, w))],
        out_specs=pl.BlockSpec((TBLK, 2 * HIDDEN), lambda w: (w, 0)),
        out_shape=jax.ShapeDtypeStruct((VOCAB_PAD, 2 * HIDDEN), jnp.float32),
    )(tt4)


def _sc_gather(x, t2, g):
    mesh = plsc.VectorSubcoreMesh(core_axis_name="c", subcore_axis_name="s")

    @functools.partial(
        pl.kernel,
        out_type=jax.ShapeDtypeStruct((2 * BATCH, 2 * HIDDEN), jnp.float32),
        mesh=mesh,
        scratch_types=[
            pltpu.VMEM((BPW * X_COLS,), jnp.float32),
            pltpu.VMEM((NCHUNK, CHUNK), jnp.int32),
            pltpu.VMEM((2, CHUNK, 2 * HIDDEN), jnp.float32),
            pltpu.SemaphoreType.DMA,
            pltpu.SemaphoreType.DMA,
            pltpu.SemaphoreType.DMA,
            pltpu.SemaphoreType.DMA,
        ],
        compiler_params=pltpu.CompilerParams(
            needs_layout_passes=False, use_tc_tiling_on_sc=False
        ),
    )
    def k(x_hbm, tab_hbm, out_hbm, x_v, idx_v, rows_v, sg0, sg1, so0, so1):
        wid = lax.axis_index("s") * NC + lax.axis_index("c")
        base = wid * BPW

        pltpu.sync_copy(x_hbm.at[pl.ds(base * X_COLS, BPW * X_COLS)], x_v)

        lane = lax.iota(jnp.int32, LANES)

        @pl.loop(0, NVEC)
        def _(v):
            p = v * LANES + lane
            q = p >> 9
            b = p & (BPW - 1)
            val = plsc.load_gather(x_v, [b * X_COLS + (2 * g + q)])
            idx_v[v >> 3, pl.ds((v & 7) * LANES, LANES)] = val.astype(jnp.int32)

        sg = (sg0, sg1)
        so = (so0, so1)

        def out_off(gg):
            return (gg >> 2) * BATCH + base + (gg & (SUBC - 1)) * CHUNK

        def g_start(gg, s):
            pltpu.async_copy(tab_hbm.at[idx_v.at[gg]], rows_v.at[s], sg[s])

        def g_wait(s):
            pltpu.make_async_copy(
                tab_hbm.at[idx_v.at[0]], rows_v.at[s], sg[s]).wait()

        def o_start(gg, s):
            pltpu.async_copy(
                rows_v.at[s], out_hbm.at[pl.ds(out_off(gg), CHUNK)], so[s])

        def o_wait(s):
            pltpu.make_async_copy(
                rows_v.at[s], out_hbm.at[pl.ds(0, CHUNK)], so[s]).wait()

        g_start(0, 0)
        g_start(1, 1)

        @pl.loop(0, NCHUNK - 2, step=2)
        def _(gg):
            for s in (0, 1):
                g_wait(s)
                o_start(gg + s, s)
                o_wait(s)
                g_start(gg + s + 2, s)

        for s, gg in ((0, NCHUNK - 2), (1, NCHUNK - 1)):
            g_wait(s)
            o_start(gg, s)
        for s in (0, 1):
            o_wait(s)

    return k(x, t2)


def _tc_project_first(g3, w3, xs, wl, bl2, wd2, bd2):
    BM = 2048

    def body(g_ref, w_ref, xs_ref, wl_ref, bl_ref, wd2_ref, bd_ref, o_ref):
        q = pl.program_id(1)

        @pl.when(q == 0)
        def _():
            scal = (
                jnp.dot(xs_ref[...], wl_ref[...],
                        preferred_element_type=jnp.float32)
                + bl_ref[...]
            )
            o_ref[...] = (
                jnp.dot(scal, wd2_ref[...], preferred_element_type=jnp.float32)
                + bd_ref[...]
            )

        o_ref[...] += jnp.dot(
            g_ref[0], w_ref[0], preferred_element_type=jnp.float32
        )

    return pl.pallas_call(
        body,
        grid=(BATCH // BM, 2),
        in_specs=[
            pl.BlockSpec((1, BM, 2 * HIDDEN), lambda i, q: (q, i, 0)),
            pl.BlockSpec((1, 2 * HIDDEN, HIDDEN), lambda i, q: (q, 0, 0)),
            pl.BlockSpec((BM, N_SCAL), lambda i, q: (i, 0)),
            pl.BlockSpec((N_SCAL, HIDDEN), lambda i, q: (0, 0)),
            pl.BlockSpec((1, HIDDEN), lambda i, q: (0, 0)),
            pl.BlockSpec((HIDDEN, HIDDEN), lambda i, q: (0, 0)),
            pl.BlockSpec((1, HIDDEN), lambda i, q: (0, 0)),
        ],
        out_specs=pl.BlockSpec((BM, HIDDEN), lambda i, q: (i, 0)),
        out_shape=jax.ShapeDtypeStruct((BATCH, HIDDEN), jnp.float32),
    )(g3, w3, xs, wl, bl2, wd2, bd2)


def _tc_project_next(g3, w3, prev):
    BM = 2048

    def body(g_ref, w_ref, p_ref, o_ref):
        q = pl.program_id(1)

        @pl.when(q == 0)
        def _():
            o_ref[...] = p_ref[...]

        o_ref[...] += jnp.dot(
            g_ref[0], w_ref[0], preferred_element_type=jnp.float32
        )

    return pl.pallas_call(
        body,
        grid=(BATCH // BM, 2),
        in_specs=[
            pl.BlockSpec((1, BM, 2 * HIDDEN), lambda i, q: (q, i, 0)),
            pl.BlockSpec((1, 2 * HIDDEN, HIDDEN), lambda i, q: (q, 0, 0)),
            pl.BlockSpec((BM, HIDDEN), lambda i, q: (i, 0)),
        ],
        out_specs=pl.BlockSpec((BM, HIDDEN), lambda i, q: (i, 0)),
        out_shape=jax.ShapeDtypeStruct((BATCH, HIDDEN), jnp.float32),
    )(g3, w3, prev)


def kernel(x, tables, Wl, bl, Wd, bd):
    tt = tables.transpose(0, 2, 1)
    tt4 = tt.reshape(NPAIR, 2 * HIDDEN, VOCAB)
    x_flat = x.reshape(-1)

    wd1 = Wd[: N_CAT * HIDDEN].reshape(N_CAT, HIDDEN, HIDDEN)
    w3 = jnp.zeros((N_CAT, 2, HIDDEN, HIDDEN), jnp.float32)
    w3 = w3.at[jnp.arange(N_CAT), jnp.arange(N_CAT) % 2].set(wd1)
    w3 = w3.reshape(N_CAT, 2 * HIDDEN, HIDDEN)
    xs = x[:, N_CAT:]
    wd2 = Wd[N_CAT * HIDDEN :]

    acc = None
    for g in range(NPAIR):
        t2 = _tc_transpose(tt4, g)
        gathered = _sc_gather(x_flat, t2, g)
        g3 = gathered.reshape(2, BATCH, 2 * HIDDEN)
        w3g = w3[2 * g : 2 * g + 2]
        if acc is None:
            acc = _tc_project_first(
                g3, w3g, xs, Wl, bl.reshape(1, HIDDEN), wd2,
                bd.reshape(1, HIDDEN),
            )
        else:
            acc = _tc_project_next(g3, w3g, acc)
    return acc


# speedup vs baseline: 2.6083x; 1.0883x over previous
import functools

import jax
import jax.numpy as jnp
from jax import lax
from jax.experimental import pallas as pl
from jax.experimental.pallas import tpu as pltpu
from jax.experimental.pallas import tpu_sc as plsc

HIDDEN = 64
N_CAT = 8
VOCAB = 100000
N_SCAL = 13
BATCH = 16384

TBLK = 8192
NTBLK = -(-VOCAB // TBLK)
VOCAB_PAD = NTBLK * TBLK
NPAIR = N_CAT // 2

NC, NS, LANES = 2, 16, 16
NW = NC * NS
BPW = BATCH // NW
RPW = BPW * 2
CHUNK = 128
NCHUNK = RPW // CHUNK
SUBC = BPW // CHUNK
NVEC = RPW // LANES

X_COLS = N_CAT + N_SCAL


def _tc_transpose(tt4, g):
    def body(a_ref, out_ref):
        i = lax.broadcasted_iota(jnp.int32, (2 * HIDDEN, 2 * HIDDEN), 0)
        j = lax.broadcasted_iota(jnp.int32, (2 * HIDDEN, 2 * HIDDEN), 1)
        eye = (i == j).astype(jnp.float32)
        out_ref[...] = lax.dot_general(
            a_ref[0], eye, (((0,), (0,)), ((), ())),
            preferred_element_type=jnp.float32,
        )

    return pl.pallas_call(
        body,
        grid=(NTBLK,),
        in_specs=[pl.BlockSpec((1, 2 * HIDDEN, TBLK), lambda w: (g, 0, w))],
        out_specs=pl.BlockSpec((TBLK, 2 * HIDDEN), lambda w: (w, 0)),
        out_shape=jax.ShapeDtypeStruct((VOCAB_PAD, 2 * HIDDEN), jnp.float32),
    )(tt4)


def _sc_gather(x, t2, g):
    mesh = plsc.VectorSubcoreMesh(core_axis_name="c", subcore_axis_name="s")

    @functools.partial(
        pl.kernel,
        out_type=jax.ShapeDtypeStruct((2 * BATCH, 2 * HIDDEN), jnp.float32),
        mesh=mesh,
        scratch_types=[
            pltpu.VMEM((BPW * X_COLS,), jnp.float32),
            pltpu.VMEM((NCHUNK, CHUNK), jnp.int32),
            pltpu.VMEM((2, CHUNK, 2 * HIDDEN), jnp.float32),
            pltpu.SemaphoreType.DMA,
            pltpu.SemaphoreType.DMA,
            pltpu.SemaphoreType.DMA,
            pltpu.SemaphoreType.DMA,
        ],
        compiler_params=pltpu.CompilerParams(
            needs_layout_passes=False, use_tc_tiling_on_sc=False
        ),
    )
    def k(x_hbm, tab_hbm, out_hbm, x_v, idx_v, rows_v, sg0, sg1, so0, so1):
        wid = lax.axis_index("s") * NC + lax.axis_index("c")
        base = wid * BPW

        pltpu.sync_copy(x_hbm.at[pl.ds(base * X_COLS, BPW * X_COLS)], x_v)

        lane = lax.iota(jnp.int32, LANES)

        @pl.loop(0, NVEC)
        def _(v):
            p = v * LANES + lane
            q = p >> 9
            b = p & (BPW - 1)
            val = plsc.load_gather(x_v, [b * X_COLS + (2 * g + q)])
            idx_v[v >> 3, pl.ds((v & 7) * LANES, LANES)] = val.astype(jnp.int32)

        sg = (sg0, sg1)
        so = (so0, so1)

        def out_off(gg):
            return (gg >> 2) * BATCH + base + (gg & (SUBC - 1)) * CHUNK

        def g_start(gg, s):
            pltpu.async_copy(tab_hbm.at[idx_v.at[gg]], rows_v.at[s], sg[s])

        def g_wait(s):
            pltpu.make_async_copy(
                tab_hbm.at[idx_v.at[0]], rows_v.at[s], sg[s]).wait()

        def o_start(gg, s):
            pltpu.async_copy(
                rows_v.at[s], out_hbm.at[pl.ds(out_off(gg), CHUNK)], so[s])

        def o_wait(s):
            pltpu.make_async_copy(
                rows_v.at[s], out_hbm.at[pl.ds(0, CHUNK)], so[s]).wait()

        g_start(0, 0)
        g_start(1, 1)

        @pl.loop(0, NCHUNK - 2, step=2)
        def _(gg):
            for s in (0, 1):
                g_wait(s)
                o_start(gg + s, s)
                o_wait(s)
                g_start(gg + s + 2, s)

        for s, gg in ((0, NCHUNK - 2), (1, NCHUNK - 1)):
            g_wait(s)
            o_start(gg, s)
        for s in (0, 1):
            o_wait(s)

    return k(x, t2)


def _tc_project_first(g3, w3, xs, wl, bl2, wd2, bd2):
    BM = 8192

    def body(g_ref, w_ref, xs_ref, wl_ref, bl_ref, wd2_ref, bd_ref, o_ref):
        q = pl.program_id(1)

        @pl.when(q == 0)
        def _():
            scal = (
                jnp.dot(xs_ref[...], wl_ref[...],
                        preferred_element_type=jnp.float32)
                + bl_ref[...]
            )
            o_ref[...] = (
                jnp.dot(scal, wd2_ref[...], preferred_element_type=jnp.float32)
                + bd_ref[...]
            )

        o_ref[...] += jnp.dot(
            g_ref[0], w_ref[0], preferred_element_type=jnp.float32
        )

    return pl.pallas_call(
        body,
        grid=(BATCH // BM, 2),
        in_specs=[
            pl.BlockSpec((1, BM, 2 * HIDDEN), lambda i, q: (q, i, 0)),
            pl.BlockSpec((1, 2 * HIDDEN, HIDDEN), lambda i, q: (q, 0, 0)),
            pl.BlockSpec((BM, N_SCAL), lambda i, q: (i, 0)),
            pl.BlockSpec((N_SCAL, HIDDEN), lambda i, q: (0, 0)),
            pl.BlockSpec((1, HIDDEN), lambda i, q: (0, 0)),
            pl.BlockSpec((HIDDEN, HIDDEN), lambda i, q: (0, 0)),
            pl.BlockSpec((1, HIDDEN), lambda i, q: (0, 0)),
        ],
        out_specs=pl.BlockSpec((BM, HIDDEN), lambda i, q: (i, 0)),
        out_shape=jax.ShapeDtypeStruct((BATCH, HIDDEN), jnp.float32),
    )(g3, w3, xs, wl, bl2, wd2, bd2)


def _tc_project_next(g3, w3, prev):
    BM = 8192

    def body(g_ref, w_ref, p_ref, o_ref):
        q = pl.program_id(1)

        @pl.when(q == 0)
        def _():
            o_ref[...] = p_ref[...]

        o_ref[...] += jnp.dot(
            g_ref[0], w_ref[0], preferred_element_type=jnp.float32
        )

    return pl.pallas_call(
        body,
        grid=(BATCH // BM, 2),
        in_specs=[
            pl.BlockSpec((1, BM, 2 * HIDDEN), lambda i, q: (q, i, 0)),
            pl.BlockSpec((1, 2 * HIDDEN, HIDDEN), lambda i, q: (q, 0, 0)),
            pl.BlockSpec((BM, HIDDEN), lambda i, q: (i, 0)),
        ],
        out_specs=pl.BlockSpec((BM, HIDDEN), lambda i, q: (i, 0)),
        out_shape=jax.ShapeDtypeStruct((BATCH, HIDDEN), jnp.float32),
    )(g3, w3, prev)


def kernel(x, tables, Wl, bl, Wd, bd):
    tt = tables.transpose(0, 2, 1)
    tt4 = tt.reshape(NPAIR, 2 * HIDDEN, VOCAB)
    x_flat = x.reshape(-1)

    wd1 = Wd[: N_CAT * HIDDEN].reshape(N_CAT, HIDDEN, HIDDEN)
    w3 = jnp.zeros((N_CAT, 2, HIDDEN, HIDDEN), jnp.float32)
    w3 = w3.at[jnp.arange(N_CAT), jnp.arange(N_CAT) % 2].set(wd1)
    w3 = w3.reshape(N_CAT, 2 * HIDDEN, HIDDEN)
    xs = x[:, N_CAT:]
    wd2 = Wd[N_CAT * HIDDEN :]

    acc = None
    for g in range(NPAIR):
        t2 = _tc_transpose(tt4, g)
        gathered = _sc_gather(x_flat, t2, g)
        g3 = gathered.reshape(2, BATCH, 2 * HIDDEN)
        w3g = w3[2 * g : 2 * g + 2]
        if acc is None:
            acc = _tc_project_first(
                g3, w3g, xs, Wl, bl.reshape(1, HIDDEN), wd2,
                bd.reshape(1, HIDDEN),
            )
        else:
            acc = _tc_project_next(g3, w3g, acc)
    return acc


# speedup vs baseline: 3.0700x; 1.1770x over previous
import functools

import jax
import jax.numpy as jnp
from jax import lax
from jax.experimental import pallas as pl
from jax.experimental.pallas import tpu as pltpu
from jax.experimental.pallas import tpu_sc as plsc

HIDDEN = 64
N_CAT = 8
VOCAB = 100000
N_SCAL = 13
BATCH = 16384

TBLK = 4096
NTBLK = -(-VOCAB // TBLK)
VOCAB_PAD = NTBLK * TBLK
NQUAD = N_CAT // 4

NC, NS, LANES = 2, 16, 16
NW = NC * NS
BPW = BATCH // NW
RPW = BPW * 4
CHUNK = 128
SUBC = BPW // CHUNK
NCHUNK = RPW // CHUNK
NVEC = RPW // LANES

X_COLS = N_CAT + N_SCAL


def _tc_transpose(tt8, s):
    def body(a_ref, out_ref):
        i = lax.broadcasted_iota(jnp.int32, (4 * HIDDEN, 4 * HIDDEN), 0)
        j = lax.broadcasted_iota(jnp.int32, (4 * HIDDEN, 4 * HIDDEN), 1)
        eye = (i == j).astype(jnp.float32)
        tr = lax.dot_general(
            a_ref[0], eye, (((0,), (0,)), ((), ())),
            preferred_element_type=jnp.float32,
        )
        u = lax.bitcast_convert_type(tr, jnp.uint32)
        r = (u + 0x7FFF + ((u >> 16) & 1)) >> 16
        w01 = (r[:, 0 * HIDDEN : 1 * HIDDEN] << 16) | r[:, 1 * HIDDEN : 2 * HIDDEN]
        w23 = (r[:, 2 * HIDDEN : 3 * HIDDEN] << 16) | r[:, 3 * HIDDEN : 4 * HIDDEN]
        packed = jnp.concatenate([w01, w23], axis=-1)
        out_ref[...] = lax.bitcast_convert_type(packed, jnp.float32)

    return pl.pallas_call(
        body,
        grid=(NTBLK,),
        in_specs=[pl.BlockSpec((1, 4 * HIDDEN, TBLK), lambda w: (s, 0, w))],
        out_specs=pl.BlockSpec((TBLK, 2 * HIDDEN), lambda w: (w, 0)),
        out_shape=jax.ShapeDtypeStruct((VOCAB_PAD, 2 * HIDDEN), jnp.float32),
    )(tt8)


def _sc_gather(x, t2, s):
    mesh = plsc.VectorSubcoreMesh(core_axis_name="c", subcore_axis_name="s")

    @functools.partial(
        pl.kernel,
        out_type=jax.ShapeDtypeStruct((2 * BATCH, 2 * HIDDEN), jnp.float32),
        mesh=mesh,
        scratch_types=[
            pltpu.VMEM((BPW * X_COLS,), jnp.float32),
            pltpu.VMEM((NCHUNK, CHUNK), jnp.int32),
            pltpu.VMEM((2, CHUNK, HIDDEN), jnp.float32),
            pltpu.SemaphoreType.DMA,
            pltpu.SemaphoreType.DMA,
            pltpu.SemaphoreType.DMA,
            pltpu.SemaphoreType.DMA,
        ],
        compiler_params=pltpu.CompilerParams(
            needs_layout_passes=False, use_tc_tiling_on_sc=False
        ),
    )
    def k(x_hbm, tab_hbm, out_hbm, x_v, idx_v, rows_v, sg0, sg1, so0, so1):
        wid = lax.axis_index("s") * NC + lax.axis_index("c")
        base = wid * BPW

        pltpu.sync_copy(x_hbm.at[pl.ds(base * X_COLS, BPW * X_COLS)], x_v)

        lane = lax.iota(jnp.int32, LANES)

        @pl.loop(0, NVEC)
        def _(v):
            p = v * LANES + lane
            q = p >> 9
            b = p & (BPW - 1)
            val = plsc.load_gather(x_v, [b * X_COLS + (4 * s + q)])
            idx_v[v >> 3, pl.ds((v & 7) * LANES, LANES)] = (
                val.astype(jnp.int32) * 2 + (q >> 1)
            )

        sg = (sg0, sg1)
        so = (so0, so1)

        def g_start(gg, s_):
            pltpu.async_copy(tab_hbm.at[idx_v.at[gg]], rows_v.at[s_], sg[s_])

        def g_wait(s_):
            pltpu.make_async_copy(
                tab_hbm.at[idx_v.at[0]], rows_v.at[s_], sg[s_]).wait()

        def o_start(gg, s_):
            q = gg >> 2
            row0 = (q >> 1) * BATCH + base + (gg & (SUBC - 1)) * CHUNK
            pltpu.async_copy(
                rows_v.at[s_],
                out_hbm.at[pl.ds(row0, CHUNK), pl.ds((q & 1) * HIDDEN, HIDDEN)],
                so[s_],
            )

        def o_wait(s_):
            pltpu.make_async_copy(
                rows_v.at[s_],
                out_hbm.at[pl.ds(0, CHUNK), pl.ds(0, HIDDEN)],
                so[s_],
            ).wait()

        g_start(0, 0)
        g_start(1, 1)

        @pl.loop(0, NCHUNK - 2, step=2)
        def _(gg):
            for s_ in (0, 1):
                g_wait(s_)
                o_start(gg + s_, s_)
                o_wait(s_)
                g_start(gg + s_ + 2, s_)

        for s_, gg in ((0, NCHUNK - 2), (1, NCHUNK - 1)):
            g_wait(s_)
            o_start(gg, s_)
        for s_ in (0, 1):
            o_wait(s_)

    return k(x, t2)


def _unpack_dot(g_ref, whi_ref, wlo_ref):
    u = lax.bitcast_convert_type(g_ref[0], jnp.uint32)
    hi = lax.bitcast_convert_type(u & jnp.uint32(0xFFFF0000), jnp.float32)
    lo = lax.bitcast_convert_type(u << 16, jnp.float32)
    return jnp.dot(hi, whi_ref[0], preferred_element_type=jnp.float32) + jnp.dot(
        lo, wlo_ref[0], preferred_element_type=jnp.float32
    )


def _tc_project_first(g3, whi, wlo, xs, wl, bl2, wd2, bd2):
    BM = 8192

    def body(g_ref, whi_ref, wlo_ref, xs_ref, wl_ref, bl_ref, wd2_ref,
             bd_ref, o_ref):
        j = pl.program_id(1)

        @pl.when(j == 0)
        def _():
            scal = (
                jnp.dot(xs_ref[...], wl_ref[...],
                        preferred_element_type=jnp.float32)
                + bl_ref[...]
            )
            o_ref[...] = (
                jnp.dot(scal, wd2_ref[...], preferred_element_type=jnp.float32)
                + bd_ref[...]
            )

        o_ref[...] += _unpack_dot(g_ref, whi_ref, wlo_ref)

    return pl.pallas_call(
        body,
        grid=(BATCH // BM, 2),
        in_specs=[
            pl.BlockSpec((1, BM, 2 * HIDDEN), lambda i, j: (j, i, 0)),
            pl.BlockSpec((1, 2 * HIDDEN, HIDDEN), lambda i, j: (j, 0, 0)),
            pl.BlockSpec((1, 2 * HIDDEN, HIDDEN), lambda i, j: (j, 0, 0)),
            pl.BlockSpec((BM, N_SCAL), lambda i, j: (i, 0)),
            pl.BlockSpec((N_SCAL, HIDDEN), lambda i, j: (0, 0)),
            pl.BlockSpec((1, HIDDEN), lambda i, j: (0, 0)),
            pl.BlockSpec((HIDDEN, HIDDEN), lambda i, j: (0, 0)),
            pl.BlockSpec((1, HIDDEN), lambda i, j: (0, 0)),
        ],
        out_specs=pl.BlockSpec((BM, HIDDEN), lambda i, j: (i, 0)),
        out_shape=jax.ShapeDtypeStruct((BATCH, HIDDEN), jnp.float32),
    )(g3, whi, wlo, xs, wl, bl2, wd2, bd2)


def _tc_project_next(g3, whi, wlo, prev):
    BM = 8192

    def body(g_ref, whi_ref, wlo_ref, p_ref, o_ref):
        j = pl.program_id(1)

        @pl.when(j == 0)
        def _():
            o_ref[...] = p_ref[...]

        o_ref[...] += _unpack_dot(g_ref, whi_ref, wlo_ref)

    return pl.pallas_call(
        body,
        grid=(BATCH // BM, 2),
        in_specs=[
            pl.BlockSpec((1, BM, 2 * HIDDEN), lambda i, j: (j, i, 0)),
            pl.BlockSpec((1, 2 * HIDDEN, HIDDEN), lambda i, j: (j, 0, 0)),
            pl.BlockSpec((1, 2 * HIDDEN, HIDDEN), lambda i, j: (j, 0, 0)),
            pl.BlockSpec((BM, HIDDEN), lambda i, j: (i, 0)),
        ],
        out_specs=pl.BlockSpec((BM, HIDDEN), lambda i, j: (i, 0)),
        out_shape=jax.ShapeDtypeStruct((BATCH, HIDDEN), jnp.float32),
    )(g3, whi, wlo, prev)


def kernel(x, tables, Wl, bl, Wd, bd):
    tt = tables.transpose(0, 2, 1)
    tt8 = tt.reshape(NQUAD, 4 * HIDDEN, VOCAB)
    x_flat = x.reshape(-1)

    wd1 = Wd[: N_CAT * HIDDEN].reshape(NQUAD, 2, 2, HIDDEN, HIDDEN)
    zed = jnp.zeros_like(wd1[:, :, 0])
    whi = jnp.concatenate([wd1[:, :, 0], zed], axis=2)
    wlo = jnp.concatenate([zed, wd1[:, :, 1]], axis=2)

    xs = x[:, N_CAT:]
    wd2 = Wd[N_CAT * HIDDEN :]

    acc = None
    for s in range(NQUAD):
        t2 = _tc_transpose(tt8, s)
        tab = t2.reshape(2 * VOCAB_PAD, HIDDEN)
        gathered = _sc_gather(x_flat, tab, s)
        g3 = gathered.reshape(2, BATCH, 2 * HIDDEN)
        if acc is None:
            acc = _tc_project_first(
                g3, whi[s], wlo[s], xs, Wl, bl.reshape(1, HIDDEN), wd2,
                bd.reshape(1, HIDDEN),
            )
        else:
            acc = _tc_project_next(g3, whi[s], wlo[s], acc)
    return acc


# speedup vs baseline: 3.1822x; 1.0365x over previous
import functools

import jax
import jax.numpy as jnp
from jax import lax
from jax.experimental import pallas as pl
from jax.experimental.pallas import tpu as pltpu
from jax.experimental.pallas import tpu_sc as plsc

HIDDEN = 64
N_CAT = 8
VOCAB = 100000
N_SCAL = 13
BATCH = 16384

TBLK = 8192
NTBLK = -(-VOCAB // TBLK)
VOCAB_PAD = NTBLK * TBLK
NQUAD = N_CAT // 4

NC, NS, LANES = 2, 16, 16
NW = NC * NS
BPW = BATCH // NW
RPW = BPW * 4
CHUNK = 128
SUBC = BPW // CHUNK
NCHUNK = RPW // CHUNK
NVEC = RPW // LANES

X_COLS = N_CAT + N_SCAL


def _tc_transpose(tt8, s):
    def body(a_ref, out_ref):
        i = lax.broadcasted_iota(jnp.int32, (4 * HIDDEN, 4 * HIDDEN), 0)
        j = lax.broadcasted_iota(jnp.int32, (4 * HIDDEN, 4 * HIDDEN), 1)
        eye = (i == j).astype(jnp.float32)
        tr = lax.dot_general(
            a_ref[0], eye, (((0,), (0,)), ((), ())),
            preferred_element_type=jnp.float32,
        )
        u = lax.bitcast_convert_type(tr, jnp.uint32)
        r = (u + 0x7FFF + ((u >> 16) & 1)) >> 16
        w01 = (r[:, 0 * HIDDEN : 1 * HIDDEN] << 16) | r[:, 1 * HIDDEN : 2 * HIDDEN]
        w23 = (r[:, 2 * HIDDEN : 3 * HIDDEN] << 16) | r[:, 3 * HIDDEN : 4 * HIDDEN]
        packed = jnp.concatenate([w01, w23], axis=-1)
        out_ref[...] = lax.bitcast_convert_type(packed, jnp.float32)

    return pl.pallas_call(
        body,
        grid=(NTBLK,),
        in_specs=[pl.BlockSpec((1, 4 * HIDDEN, TBLK), lambda w: (s, 0, w))],
        out_specs=pl.BlockSpec((TBLK, 2 * HIDDEN), lambda w: (w, 0)),
        out_shape=jax.ShapeDtypeStruct((VOCAB_PAD, 2 * HIDDEN), jnp.float32),
    )(tt8)


def _sc_gather(x, t2, s):
    mesh = plsc.VectorSubcoreMesh(core_axis_name="c", subcore_axis_name="s")

    @functools.partial(
        pl.kernel,
        out_type=jax.ShapeDtypeStruct((2 * BATCH, 2 * HIDDEN), jnp.float32),
        mesh=mesh,
        scratch_types=[
            pltpu.VMEM((BPW * X_COLS,), jnp.float32),
            pltpu.VMEM((NCHUNK, CHUNK), jnp.int32),
            pltpu.VMEM((2, CHUNK, HIDDEN), jnp.float32),
            pltpu.SemaphoreType.DMA,
            pltpu.SemaphoreType.DMA,
            pltpu.SemaphoreType.DMA,
            pltpu.SemaphoreType.DMA,
        ],
        compiler_params=pltpu.CompilerParams(
            needs_layout_passes=False, use_tc_tiling_on_sc=False
        ),
    )
    def k(x_hbm, tab_hbm, out_hbm, x_v, idx_v, rows_v, sg0, sg1, so0, so1):
        wid = lax.axis_index("s") * NC + lax.axis_index("c")
        base = wid * BPW

        pltpu.sync_copy(x_hbm.at[pl.ds(base * X_COLS, BPW * X_COLS)], x_v)

        lane = lax.iota(jnp.int32, LANES)

        @pl.loop(0, NVEC)
        def _(v):
            p = v * LANES + lane
            q = p >> 9
            b = p & (BPW - 1)
            val = plsc.load_gather(x_v, [b * X_COLS + (4 * s + q)])
            idx_v[v >> 3, pl.ds((v & 7) * LANES, LANES)] = (
                val.astype(jnp.int32) * 2 + (q >> 1)
            )

        sg = (sg0, sg1)
        so = (so0, so1)

        def g_start(gg, s_):
            pltpu.async_copy(tab_hbm.at[idx_v.at[gg]], rows_v.at[s_], sg[s_])

        def g_wait(s_):
            pltpu.make_async_copy(
                tab_hbm.at[idx_v.at[0]], rows_v.at[s_], sg[s_]).wait()

        def o_start(gg, s_):
            q = gg >> 2
            row0 = (q >> 1) * BATCH + base + (gg & (SUBC - 1)) * CHUNK
            pltpu.async_copy(
                rows_v.at[s_],
                out_hbm.at[pl.ds(row0, CHUNK), pl.ds((q & 1) * HIDDEN, HIDDEN)],
                so[s_],
            )

        def o_wait(s_):
            pltpu.make_async_copy(
                rows_v.at[s_],
                out_hbm.at[pl.ds(0, CHUNK), pl.ds(0, HIDDEN)],
                so[s_],
            ).wait()

        g_start(0, 0)
        g_start(1, 1)

        @pl.loop(0, NCHUNK - 2, step=2)
        def _(gg):
            for s_ in (0, 1):
                g_wait(s_)
                o_start(gg + s_, s_)
                o_wait(s_)
                g_start(gg + s_ + 2, s_)

        for s_, gg in ((0, NCHUNK - 2), (1, NCHUNK - 1)):
            g_wait(s_)
            o_start(gg, s_)
        for s_ in (0, 1):
            o_wait(s_)

    return k(x, t2)


def _unpack_dot(g_ref, whi_ref, wlo_ref):
    u = lax.bitcast_convert_type(g_ref[0], jnp.uint32)
    hi = lax.bitcast_convert_type(u & jnp.uint32(0xFFFF0000), jnp.float32)
    lo = lax.bitcast_convert_type(u << 16, jnp.float32)
    return jnp.dot(hi, whi_ref[0], preferred_element_type=jnp.float32) + jnp.dot(
        lo, wlo_ref[0], preferred_element_type=jnp.float32
    )


def _tc_project_first(g3, whi, wlo, xs, wl, bl2, wd2, bd2):
    BM = 8192

    def body(g_ref, whi_ref, wlo_ref, xs_ref, wl_ref, bl_ref, wd2_ref,
             bd_ref, o_ref):
        j = pl.program_id(1)

        @pl.when(j == 0)
        def _():
            scal = (
                jnp.dot(xs_ref[...], wl_ref[...],
                        preferred_element_type=jnp.float32)
                + bl_ref[...]
            )
            o_ref[...] = (
                jnp.dot(scal, wd2_ref[...], preferred_element_type=jnp.float32)
                + bd_ref[...]
            )

        o_ref[...] += _unpack_dot(g_ref, whi_ref, wlo_ref)

    return pl.pallas_call(
        body,
        grid=(BATCH // BM, 2),
        in_specs=[
            pl.BlockSpec((1, BM, 2 * HIDDEN), lambda i, j: (j, i, 0)),
            pl.BlockSpec((1, 2 * HIDDEN, HIDDEN), lambda i, j: (j, 0, 0)),
            pl.BlockSpec((1, 2 * HIDDEN, HIDDEN), lambda i, j: (j, 0, 0)),
            pl.BlockSpec((BM, N_SCAL), lambda i, j: (i, 0)),
            pl.BlockSpec((N_SCAL, HIDDEN), lambda i, j: (0, 0)),
            pl.BlockSpec((1, HIDDEN), lambda i, j: (0, 0)),
            pl.BlockSpec((HIDDEN, HIDDEN), lambda i, j: (0, 0)),
            pl.BlockSpec((1, HIDDEN), lambda i, j: (0, 0)),
        ],
        out_specs=pl.BlockSpec((BM, HIDDEN), lambda i, j: (i, 0)),
        out_shape=jax.ShapeDtypeStruct((BATCH, HIDDEN), jnp.float32),
    )(g3, whi, wlo, xs, wl, bl2, wd2, bd2)


def _tc_project_next(g3, whi, wlo, prev):
    BM = 8192

    def body(g_ref, whi_ref, wlo_ref, p_ref, o_ref):
        j = pl.program_id(1)

        @pl.when(j == 0)
        def _():
            o_ref[...] = p_ref[...]

        o_ref[...] += _unpack_dot(g_ref, whi_ref, wlo_ref)

    return pl.pallas_call(
        body,
        grid=(BATCH // BM, 2),
        in_specs=[
            pl.BlockSpec((1, BM, 2 * HIDDEN), lambda i, j: (j, i, 0)),
            pl.BlockSpec((1, 2 * HIDDEN, HIDDEN), lambda i, j: (j, 0, 0)),
            pl.BlockSpec((1, 2 * HIDDEN, HIDDEN), lambda i, j: (j, 0, 0)),
            pl.BlockSpec((BM, HIDDEN), lambda i, j: (i, 0)),
        ],
        out_specs=pl.BlockSpec((BM, HIDDEN), lambda i, j: (i, 0)),
        out_shape=jax.ShapeDtypeStruct((BATCH, HIDDEN), jnp.float32),
    )(g3, whi, wlo, prev)


def kernel(x, tables, Wl, bl, Wd, bd):
    tt = tables.transpose(0, 2, 1)
    tt8 = tt.reshape(NQUAD, 4 * HIDDEN, VOCAB)
    x_flat = x.reshape(-1)

    wd1 = Wd[: N_CAT * HIDDEN].reshape(NQUAD, 2, 2, HIDDEN, HIDDEN)
    zed = jnp.zeros_like(wd1[:, :, 0])
    whi = jnp.concatenate([wd1[:, :, 0], zed], axis=2)
    wlo = jnp.concatenate([zed, wd1[:, :, 1]], axis=2)

    xs = x[:, N_CAT:]
    wd2 = Wd[N_CAT * HIDDEN :]

    acc = None
    for s in range(NQUAD):
        t2 = _tc_transpose(tt8, s)
        tab = t2.reshape(2 * VOCAB_PAD, HIDDEN)
        gathered = _sc_gather(x_flat, tab, s)
        g3 = gathered.reshape(2, BATCH, 2 * HIDDEN)
        if acc is None:
            acc = _tc_project_first(
                g3, whi[s], wlo[s], xs, Wl, bl.reshape(1, HIDDEN), wd2,
                bd.reshape(1, HIDDEN),
            )
        else:
            acc = _tc_project_next(g3, whi[s], wlo[s], acc)
    return acc
